# baseline (speedup 1.0000x reference)
import functools

import jax
import jax.numpy as jnp
from jax import lax
from jax.experimental import pallas as pl
from jax.experimental.pallas import tpu as pltpu
from jax.experimental.pallas import tpu_sc as plsc

D_MODEL = 64
SCALE = 8.0
NC = 2
NS = 16
NW = NC * NS
C = 128
NBUF = 4


@functools.lru_cache(maxsize=None)
def _make_emb(N, S):
    ntb = N // C
    nch = N * S // (C * NW)
    nchj = nch // NBUF
    mesh = plsc.VectorSubcoreMesh(core_axis_name="c", subcore_axis_name="s")

    @functools.partial(
        pl.kernel,
        out_type=jax.ShapeDtypeStruct((S, 8, ntb, 8, C), jnp.float32),
        mesh=mesh,
        scratch_types=(
            [pltpu.VMEM((nch, C), jnp.int32)]
            + [pltpu.VMEM((C, D_MODEL), jnp.float32)] * NBUF
            + [pltpu.VMEM((8, 8, C), jnp.float32)] * NBUF
            + [pltpu.SemaphoreType.DMA] * (2 * NBUF)
        ),
        compiler_params=pltpu.CompilerParams(
            use_tc_tiling_on_sc=False, needs_layout_passes=False),
    )
    def emb(x_hbm, table_hbm, out_hbm, idx_v, *bufs):
        gbuf = bufs[0:NBUF]
        obuf = bufs[NBUF:2 * NBUF]
        gsem = bufs[2 * NBUF:3 * NBUF]
        osem = bufs[3 * NBUF:4 * NBUF]
        wid = lax.axis_index("s") * NC + lax.axis_index("c")
        pltpu.sync_copy(x_hbm.at[wid], idx_v)
        lane = lax.iota(jnp.int32, 16)

        for b in range(NBUF):
            pltpu.async_copy(table_hbm.at[idx_v.at[b]], gbuf[b], gsem[b])

        def body(j, carry):
            for b in range(NBUF):
                g = j * NBUF + b
                f = wid * nch + g
                s = f // ntb
                nb = f % ntb
                pltpu.make_async_copy(
                    table_hbm.at[idx_v.at[0]], gbuf[b], gsem[b]).wait()

                @pl.when(j > 0)
                def _():
                    pltpu.make_async_copy(
                        obuf[b], out_hbm.at[0, :, 0], osem[b]).wait()

                src, dst = gbuf[b], obuf[b]

                @plsc.parallel_loop(0, D_MODEL * C // 16, 1, unroll=8)
                def _tsc(v):
                    d = v // 8
                    c16 = (v % 8) * 16
                    vals = plsc.load_gather(
                        src, [c16 + lane, jnp.full((16,), d, jnp.int32)])
                    dst[d // 8, d % 8, pl.ds(c16, 16)] = vals * SCALE

                @pl.when(j < nchj - 1)
                def _():
                    pltpu.async_copy(
                        table_hbm.at[idx_v.at[g + NBUF]], gbuf[b], gsem[b])

                pltpu.async_copy(obuf[b], out_hbm.at[s, :, nb], osem[b])
            return carry

        lax.fori_loop(0, nchj, body, 0)
        for b in range(NBUF):
            pltpu.make_async_copy(
                obuf[b], out_hbm.at[0, :, 0], osem[b]).wait()

    return emb


def kernel(x, table):
    n, s = x.shape
    idx = x.T.reshape(NW, n * s // (NW * C), C).astype(jnp.int32)
    out5 = _make_emb(n, s)(idx, table)
    return out5.transpose(2, 4, 0, 1, 3).reshape(n, s, D_MODEL)


# speedup vs baseline: 1.2158x; 1.2158x over previous
import functools

import jax
import jax.numpy as jnp
from jax import lax
from jax.experimental import pallas as pl
from jax.experimental.pallas import tpu as pltpu
from jax.experimental.pallas import tpu_sc as plsc

D_MODEL = 64
SCALE = 8.0
NC = 2
NS = 16
NW = NC * NS
C = 128
NBUF = 4


@functools.lru_cache(maxsize=None)
def _make_emb(N, S):
    ntb = N // C
    nch = N * S // (C * NW)
    nchj = nch // NBUF
    mesh = plsc.VectorSubcoreMesh(core_axis_name="c", subcore_axis_name="s")

    @functools.partial(
        pl.kernel,
        out_type=jax.ShapeDtypeStruct((S, 8, ntb, 8, C), jnp.float32),
        mesh=mesh,
        scratch_types=(
            [pltpu.VMEM((nch, C), jnp.int32)]
            + [pltpu.VMEM((C, D_MODEL), jnp.float32)] * NBUF
            + [pltpu.VMEM((8, 8, C), jnp.float32)] * NBUF
            + [pltpu.SemaphoreType.DMA] * (2 * NBUF)
        ),
        compiler_params=pltpu.CompilerParams(
            use_tc_tiling_on_sc=False, needs_layout_passes=False),
    )
    def emb(x_hbm, table_hbm, out_hbm, idx_v, *bufs):
        gbuf = bufs[0:NBUF]
        obuf = bufs[NBUF:2 * NBUF]
        gsem = bufs[2 * NBUF:3 * NBUF]
        osem = bufs[3 * NBUF:4 * NBUF]
        wid = lax.axis_index("s") * NC + lax.axis_index("c")
        pltpu.sync_copy(x_hbm.at[wid], idx_v)
        lane = lax.iota(jnp.int32, 16)
        rowvs = [lane + 16 * k for k in range(8)]

        for b in range(NBUF):
            pltpu.async_copy(table_hbm.at[idx_v.at[b]], gbuf[b], gsem[b])

        def body(j, carry):
            for b in range(NBUF):
                g = j * NBUF + b
                f = wid * nch + g
                s = f // ntb
                nb = f % ntb
                pltpu.make_async_copy(
                    table_hbm.at[idx_v.at[0]], gbuf[b], gsem[b]).wait()

                @pl.when(j > 0)
                def _():
                    pltpu.make_async_copy(
                        obuf[b], out_hbm.at[0, :, 0], osem[b]).wait()

                src, dst = gbuf[b], obuf[b]

                @plsc.parallel_loop(0, D_MODEL, 1, unroll=4)
                def _tsc(d):
                    colv = jnp.full((16,), d, jnp.int32)
                    tr = d // 8
                    r = d % 8
                    for k in range(8):
                        vals = plsc.load_gather(src, [rowvs[k], colv])
                        dst[tr, r, pl.ds(16 * k, 16)] = vals * SCALE

                @pl.when(j < nchj - 1)
                def _():
                    pltpu.async_copy(
                        table_hbm.at[idx_v.at[g + NBUF]], gbuf[b], gsem[b])

                pltpu.async_copy(obuf[b], out_hbm.at[s, :, nb], osem[b])
            return carry

        lax.fori_loop(0, nchj, body, 0)
        for b in range(NBUF):
            pltpu.make_async_copy(
                obuf[b], out_hbm.at[0, :, 0], osem[b]).wait()

    return emb


def kernel(x, table):
    n, s = x.shape
    idx = x.T.reshape(NW, n * s // (NW * C), C).astype(jnp.int32)
    out5 = _make_emb(n, s)(idx, table)
    return out5.transpose(2, 4, 0, 1, 3).reshape(n, s, D_MODEL)


# speedup vs baseline: 1.7506x; 1.4399x over previous
import functools

import jax
import jax.numpy as jnp
from jax import lax
from jax.experimental import pallas as pl
from jax.experimental.pallas import tpu as pltpu
from jax.experimental.pallas import tpu_sc as plsc

D_MODEL = 64
SCALE = 8.0
NC = 2
NS = 16
NW = NC * NS
C = 128
NBUF = 4


@functools.lru_cache(maxsize=None)
def _make_emb(N, S):
    ntb = N // C
    nch = N * S // (C * NW)
    nchj = nch // NBUF
    mesh = plsc.VectorSubcoreMesh(core_axis_name="c", subcore_axis_name="s")

    @functools.partial(
        pl.kernel,
        out_type=jax.ShapeDtypeStruct((S, 8, ntb, 8, C), jnp.float32),
        mesh=mesh,
        scratch_types=(
            [pltpu.VMEM((nch, C), jnp.int32)]
            + [pltpu.VMEM((C, 2 * D_MODEL), jnp.float32)] * NBUF
            + [pltpu.VMEM((8, 8, C), jnp.float32)] * NBUF
            + [pltpu.VMEM((C,), jnp.int32)] * NBUF
            + [pltpu.SemaphoreType.DMA] * (2 * NBUF)
        ),
        compiler_params=pltpu.CompilerParams(
            use_tc_tiling_on_sc=True, needs_layout_passes=False),
    )
    def emb(x_hbm, table_hbm, out_hbm, idx_v, *bufs):
        gbuf = bufs[0:NBUF]
        obuf = bufs[NBUF:2 * NBUF]
        hbuf = bufs[2 * NBUF:3 * NBUF]
        gsem = bufs[3 * NBUF:4 * NBUF]
        osem = bufs[4 * NBUF:5 * NBUF]
        wid = lax.axis_index("s") * NC + lax.axis_index("c")
        pltpu.sync_copy(x_hbm.at[wid], idx_v)
        lane = lax.iota(jnp.int32, 16)
        rowvs = [lane + 16 * k for k in range(8)]

        def halve(g, b):
            for m in range(8):
                sl = pl.ds(16 * m, 16)
                hbuf[b][sl] = jnp.right_shift(idx_v[g, sl], 1)

        for b in range(NBUF):
            halve(b, b)
            pltpu.async_copy(table_hbm.at[hbuf[b]], gbuf[b], gsem[b])

        def body(j, carry):
            for b in range(NBUF):
                g = j * NBUF + b
                f = wid * nch + g
                s = f // ntb
                nb = f % ntb
                pltpu.make_async_copy(
                    table_hbm.at[hbuf[b]], gbuf[b], gsem[b]).wait()

                @pl.when(j > 0)
                def _():
                    pltpu.make_async_copy(
                        obuf[b], out_hbm.at[0, :, 0], osem[b]).wait()

                src, dst = gbuf[b], obuf[b]
                pars = [
                    jnp.left_shift(
                        jnp.bitwise_and(idx_v[g, pl.ds(16 * k, 16)], 1), 6)
                    for k in range(8)
                ]

                @plsc.parallel_loop(0, 16, 1, unroll=2)
                def _tsc(a):
                    rot = jnp.bitwise_and(lane + a, 15)
                    for d0 in range(0, D_MODEL, 16):
                        dv = rot + d0
                        tr_v = jnp.right_shift(dv, 3)
                        r_v = jnp.bitwise_and(dv, 7)
                        for k in range(8):
                            vals = plsc.load_gather(
                                src, [rowvs[k], dv + pars[k]])
                            plsc.store_scatter(
                                dst, [tr_v, r_v, rowvs[k]], vals * SCALE)

                @pl.when(j < nchj - 1)
                def _():
                    halve(g + NBUF, b)
                    pltpu.async_copy(
                        table_hbm.at[hbuf[b]], gbuf[b], gsem[b])

                pltpu.async_copy(obuf[b], out_hbm.at[s, :, nb], osem[b])
            return carry

        lax.fori_loop(0, nchj, body, 0)
        for b in range(NBUF):
            pltpu.make_async_copy(
                obuf[b], out_hbm.at[0, :, 0], osem[b]).wait()

    return emb


def kernel(x, table):
    n, s = x.shape
    v = table.shape[0]
    idx = x.T.reshape(NW, n * s // (NW * C), C).astype(jnp.int32)
    tbl2 = table.reshape(v // 2, 2 * D_MODEL)
    out5 = _make_emb(n, s)(idx, tbl2)
    return out5.transpose(2, 4, 0, 1, 3).reshape(n, s, D_MODEL)


# speedup vs baseline: 1.8585x; 1.0617x over previous
import functools

import jax
import jax.numpy as jnp
from jax import lax
from jax.experimental import pallas as pl
from jax.experimental.pallas import tpu as pltpu
from jax.experimental.pallas import tpu_sc as plsc

D_MODEL = 64
SCALE = 8.0
NC = 2
NS = 16
NW = NC * NS
C = 128
NBUF = 4


@functools.lru_cache(maxsize=None)
def _make_emb(N, S):
    ntb = N // C
    nch = N * S // (C * NW)
    nchj = nch // NBUF
    mesh = plsc.VectorSubcoreMesh(core_axis_name="c", subcore_axis_name="s")

    @functools.partial(
        pl.kernel,
        out_type=jax.ShapeDtypeStruct((S, 8, ntb, 8, C), jnp.float32),
        mesh=mesh,
        scratch_types=(
            [pltpu.VMEM((nch, C), jnp.int32)]
            + [pltpu.VMEM((C, D_MODEL), jnp.float32)] * NBUF
            + [pltpu.VMEM((8, 8, C), jnp.float32)] * NBUF
            + [pltpu.SemaphoreType.DMA] * (2 * NBUF)
        ),
        compiler_params=pltpu.CompilerParams(
            use_tc_tiling_on_sc=False, needs_layout_passes=False),
    )
    def emb(x_hbm, table_hbm, out_hbm, idx_v, *bufs):
        gbuf = bufs[0:NBUF]
        obuf = bufs[NBUF:2 * NBUF]
        gsem = bufs[2 * NBUF:3 * NBUF]
        osem = bufs[3 * NBUF:4 * NBUF]
        wid = lax.axis_index("s") * NC + lax.axis_index("c")
        pltpu.sync_copy(x_hbm.at[wid], idx_v)
        lane = lax.iota(jnp.int32, 16)
        rowvs = [lane + 16 * k for k in range(8)]

        for b in range(NBUF):
            pltpu.async_copy(table_hbm.at[idx_v.at[b]], gbuf[b], gsem[b])

        def body(j, carry):
            for b in range(NBUF):
                g = j * NBUF + b
                f = wid * nch + g
                s = f // ntb
                nb = f % ntb
                pltpu.make_async_copy(
                    table_hbm.at[idx_v.at[0]], gbuf[b], gsem[b]).wait()

                @pl.when(j > 0)
                def _():
                    pltpu.make_async_copy(
                        obuf[b], out_hbm.at[0, :, 0], osem[b]).wait()

                src, dst = gbuf[b], obuf[b]

                @plsc.parallel_loop(0, 16, 1, unroll=2)
                def _tsc(a):
                    rot = jnp.bitwise_and(lane + a, 15)
                    for d0 in range(0, D_MODEL, 16):
                        colv = rot + d0
                        tr_v = jnp.right_shift(colv, 3)
                        r_v = jnp.bitwise_and(colv, 7)
                        for k in range(8):
                            vals = plsc.load_gather(src, [rowvs[k], colv])
                            plsc.store_scatter(
                                dst, [tr_v, r_v, rowvs[k]], vals * SCALE)

                @pl.when(j < nchj - 1)
                def _():
                    pltpu.async_copy(
                        table_hbm.at[idx_v.at[g + NBUF]], gbuf[b], gsem[b])

                pltpu.async_copy(obuf[b], out_hbm.at[s, :, nb], osem[b])
            return carry

        lax.fori_loop(0, nchj, body, 0)
        for b in range(NBUF):
            pltpu.make_async_copy(
                obuf[b], out_hbm.at[0, :, 0], osem[b]).wait()

    return emb


def kernel(x, table):
    n, s = x.shape
    idx = x.T.reshape(NW, n * s // (NW * C), C).astype(jnp.int32)
    out5 = _make_emb(n, s)(idx, table)
    return out5.transpose(2, 4, 0, 1, 3).reshape(n, s, D_MODEL)
